# baseline (speedup 1.0000x reference)
import jax
import jax.numpy as jnp
from jax import lax
from jax.experimental import pallas as pl
from jax.experimental.pallas import tpu as pltpu
from jax.experimental.pallas import tpu_sc as plsc

NN = 10000
EE = 160000
DD = 128

NCORES = 2
NSUB = 16
NTILES = NCORES * NSUB
CHUNK = 96
NCHUNKS = 53
EDGES_PER_TILE = CHUNK * NCHUNKS
EPAD = NTILES * EDGES_PER_TILE
ACC_ROWS = 10240
ZROWS = ACC_ROWS // NSUB
RING = 3


def _sc_body(keys_hbm, kprev_hbm, x_hbm, zeros_hbm, p_hbm,
             kv_v, kp_v, tidx_v, sidx_v, rows_v, accum_sh,
             ksem0, ksem1, ksem2, gsem0, gsem1, gsem2, ssem0, ssem1, ssem2):
    c = lax.axis_index("c")
    s = lax.axis_index("s")
    lane = lax.iota(jnp.int32, 16)
    ksems = [ksem0, ksem1, ksem2]
    gsems = [gsem0, gsem1, gsem2]
    ssems = [ssem0, ssem1, ssem2]

    tile_base = (c * NSUB + s) * EDGES_PER_TILE
    trash = 10000 + s

    def start_keys(k):
        b = k % RING
        e0 = tile_base + k * CHUNK
        h1 = pltpu.async_copy(keys_hbm.at[pl.ds(e0, CHUNK)], kv_v.at[b],
                              ksems[b])
        h2 = pltpu.async_copy(kprev_hbm.at[pl.ds(e0, CHUNK)], kp_v.at[b],
                              ksems[b])
        return (h1, h2)

    def compute_indices(k):
        b = k % RING
        def grp(j, carry):
            kv = kv_v[b, pl.ds(j * 16, 16)]
            kp = kp_v[b, pl.ds(j * 16, 16)]
            keep = kv != kp
            src = lax.shift_right_logical(kv, 14)
            t = lax.bitwise_and(kv, 16383)
            dummy_t = lax.bitwise_and(tile_base + k * CHUNK + j * 16 + lane,
                                      8191)
            tidx_v[b, pl.ds(j * 16, 16)] = jnp.where(keep, t, dummy_t)
            sidx_v[b, pl.ds(j * 16, 16)] = jnp.where(keep, src, trash)
            return carry
        lax.fori_loop(0, CHUNK // 16, grp, 0)

    khandles = [None] * NCHUNKS
    for k in range(RING):
        khandles[k] = start_keys(k)
    pltpu.sync_copy(zeros_hbm, accum_sh.at[pl.ds(s * ZROWS, ZROWS)])
    plsc.subcore_barrier()

    ghandles = [None] * NCHUNKS
    shandles = [None] * NCHUNKS
    for k in range(NCHUNKS):
        b = k % RING
        for h in khandles[k]:
            h.wait()
        if k >= RING:
            shandles[k - RING].wait()
        compute_indices(k)
        ghandles[k] = pltpu.async_copy(x_hbm.at[tidx_v.at[b]], rows_v.at[b],
                                       gsems[b])
        if k + RING < NCHUNKS:
            khandles[k + RING] = start_keys(k + RING)
        if k >= 1:
            bp = (k - 1) % RING
            ghandles[k - 1].wait()
            shandles[k - 1] = pltpu.async_copy(
                rows_v.at[bp], accum_sh.at[sidx_v.at[bp]], ssems[bp],
                add=True)
    ghandles[NCHUNKS - 1].wait()
    shandles[NCHUNKS - 1] = pltpu.async_copy(
        rows_v.at[(NCHUNKS - 1) % RING],
        accum_sh.at[sidx_v.at[(NCHUNKS - 1) % RING]],
        ssems[(NCHUNKS - 1) % RING], add=True)
    for k in range(NCHUNKS - RING, NCHUNKS):
        shandles[k].wait()
    plsc.subcore_barrier()

    row0 = s * 624
    for q in range(8):
        r = row0 + q * 80
        pltpu.sync_copy(accum_sh.at[pl.ds(r, 80)], rows_v.at[0, pl.ds(0, 80)])
        pltpu.sync_copy(rows_v.at[0, pl.ds(0, 80)], p_hbm.at[c, pl.ds(r, 80)])


@jax.jit
def _sc_segment_sum(keys_p, kprev_p, x, zeros_rows):
    mesh = plsc.VectorSubcoreMesh(core_axis_name="c", subcore_axis_name="s")
    kern = pl.kernel(
        _sc_body,
        out_type=jax.ShapeDtypeStruct((NCORES, NN, DD), jnp.float32),
        mesh=mesh,
        scratch_types=[
            pltpu.VMEM((RING, CHUNK), jnp.int32),
            pltpu.VMEM((RING, CHUNK), jnp.int32),
            pltpu.VMEM((RING, CHUNK), jnp.int32),
            pltpu.VMEM((RING, CHUNK), jnp.int32),
            pltpu.VMEM((RING, CHUNK, DD), jnp.float32),
            pltpu.VMEM_SHARED((ACC_ROWS, DD), jnp.float32),
            pltpu.SemaphoreType.DMA,
            pltpu.SemaphoreType.DMA,
            pltpu.SemaphoreType.DMA,
            pltpu.SemaphoreType.DMA,
            pltpu.SemaphoreType.DMA,
            pltpu.SemaphoreType.DMA,
            pltpu.SemaphoreType.DMA,
            pltpu.SemaphoreType.DMA,
            pltpu.SemaphoreType.DMA,
        ],
    )
    return kern(keys_p, kprev_p, x, zeros_rows)


def _mm_body(p_ref, w_ref, b_ref, o_ref):
    xs = p_ref[0] + p_ref[1]
    o_ref[...] = jnp.dot(xs, w_ref[...],
                         preferred_element_type=jnp.float32) + b_ref[...]


@jax.jit
def _combine_matmul(p, w, b2):
    bm = 1000
    return pl.pallas_call(
        _mm_body,
        grid=(NN // bm,),
        in_specs=[
            pl.BlockSpec((NCORES, bm, DD), lambda i: (0, i, 0)),
            pl.BlockSpec((DD, DD), lambda i: (0, 0)),
            pl.BlockSpec((1, DD), lambda i: (0, 0)),
        ],
        out_specs=pl.BlockSpec((bm, DD), lambda i: (i, 0)),
        out_shape=jax.ShapeDtypeStruct((NN, DD), jnp.float32),
    )(p, w, b2)


from jax.experimental.compute_on import compute_on


@compute_on("tpu_sparsecore")
@jax.jit
def _sc_sort(keys):
    return lax.sort(keys, is_stable=False)


def kernel(x, edge_list, W, b):
    src = edge_list[0].astype(jnp.int32)
    dst = edge_list[1].astype(jnp.int32)
    keys = src * 16384 + dst
    ks = _sc_sort(keys)
    kprev = jnp.concatenate([jnp.full((1,), -1, jnp.int32), ks[:-1]])
    zpad = jnp.zeros((EPAD - EE,), jnp.int32)
    ksp = jnp.concatenate([ks, zpad])
    kpp = jnp.concatenate([kprev, zpad])
    zeros_rows = jnp.zeros((ZROWS, DD), jnp.float32)
    partials = _sc_segment_sum(ksp, kpp, x, zeros_rows)
    return _combine_matmul(partials, W, b.reshape(1, DD))


# speedup vs baseline: 1.0312x; 1.0312x over previous
import jax
import jax.numpy as jnp
from jax import lax
from jax.experimental import pallas as pl
from jax.experimental.pallas import tpu as pltpu
from jax.experimental.pallas import tpu_sc as plsc

NN = 10000
EE = 160000
DD = 128

NCORES = 2
NSUB = 16
NTILES = NCORES * NSUB
CHUNK = 96
NCHUNKS = 53
EDGES_PER_TILE = CHUNK * NCHUNKS
EPAD = NTILES * EDGES_PER_TILE
ACC_ROWS = 10240
ZROWS = ACC_ROWS // NSUB
RING = 3


def _sc_body(keys_hbm, kprev_hbm, x_hbm, p_hbm,
             kv_v, kp_v, tidx_v, sidx_v, rows_v, accum_sh,
             ksem0, ksem1, ksem2, gsem0, gsem1, gsem2, ssem0, ssem1, ssem2):
    c = lax.axis_index("c")
    s = lax.axis_index("s")
    lane = lax.iota(jnp.int32, 16)
    ksems = [ksem0, ksem1, ksem2]
    gsems = [gsem0, gsem1, gsem2]
    ssems = [ssem0, ssem1, ssem2]

    tile_base = (c * NSUB + s) * EDGES_PER_TILE
    trash = 10000 + s

    def start_keys(k):
        b = k % RING
        e0 = tile_base + k * CHUNK
        h1 = pltpu.async_copy(keys_hbm.at[pl.ds(e0, CHUNK)], kv_v.at[b],
                              ksems[b])
        h2 = pltpu.async_copy(kprev_hbm.at[pl.ds(e0, CHUNK)], kp_v.at[b],
                              ksems[b])
        return (h1, h2)

    def compute_indices(k):
        b = k % RING
        def grp(j, carry):
            kv = kv_v[b, pl.ds(j * 16, 16)]
            kp = kp_v[b, pl.ds(j * 16, 16)]
            keep = kv != kp
            src = lax.shift_right_logical(kv, 14)
            t = lax.bitwise_and(kv, 16383)
            dummy_t = lax.bitwise_and(tile_base + k * CHUNK + j * 16 + lane,
                                      8191)
            tidx_v[b, pl.ds(j * 16, 16)] = jnp.where(keep, t, dummy_t)
            sidx_v[b, pl.ds(j * 16, 16)] = jnp.where(keep, src, trash)
            return carry
        lax.fori_loop(0, CHUNK // 16, grp, 0)

    khandles = [None] * NCHUNKS
    for k in range(RING):
        khandles[k] = start_keys(k)

    def zrow(i, carry):
        for j in range(DD // 16):
            rows_v[0, i, pl.ds(j * 16, 16)] = jnp.zeros((16,), jnp.float32)
        return carry
    lax.fori_loop(0, 80, zrow, 0)
    zhandles = []
    for q in range(8):
        zhandles.append(pltpu.async_copy(
            rows_v.at[0, pl.ds(0, 80)],
            accum_sh.at[pl.ds(s * ZROWS + q * 80, 80)], ssems[q % RING]))
    for h in zhandles:
        h.wait()
    plsc.subcore_barrier()

    ghandles = [None] * NCHUNKS
    shandles = [None] * NCHUNKS
    for k in range(NCHUNKS):
        b = k % RING
        for h in khandles[k]:
            h.wait()
        if k >= RING:
            shandles[k - RING].wait()
        compute_indices(k)
        ghandles[k] = pltpu.async_copy(x_hbm.at[tidx_v.at[b]], rows_v.at[b],
                                       gsems[b])
        if k + RING < NCHUNKS:
            khandles[k + RING] = start_keys(k + RING)
        if k >= 1:
            bp = (k - 1) % RING
            ghandles[k - 1].wait()
            shandles[k - 1] = pltpu.async_copy(
                rows_v.at[bp], accum_sh.at[sidx_v.at[bp]], ssems[bp],
                add=True)
    ghandles[NCHUNKS - 1].wait()
    shandles[NCHUNKS - 1] = pltpu.async_copy(
        rows_v.at[(NCHUNKS - 1) % RING],
        accum_sh.at[sidx_v.at[(NCHUNKS - 1) % RING]],
        ssems[(NCHUNKS - 1) % RING], add=True)
    for k in range(NCHUNKS - RING, NCHUNKS):
        shandles[k].wait()
    plsc.subcore_barrier()

    row0 = s * 624
    ehandles = []
    for q in range(8):
        r = row0 + q * 80
        ehandles.append(pltpu.async_copy(
            accum_sh.at[pl.ds(r, 80)], p_hbm.at[c, pl.ds(r, 80)],
            ssems[q % RING]))
    for h in ehandles:
        h.wait()


@jax.jit
def _sc_segment_sum(keys_p, kprev_p, x):
    mesh = plsc.VectorSubcoreMesh(core_axis_name="c", subcore_axis_name="s")
    kern = pl.kernel(
        _sc_body,
        out_type=jax.ShapeDtypeStruct((NCORES, NN, DD), jnp.float32),
        mesh=mesh,
        scratch_types=[
            pltpu.VMEM((RING, CHUNK), jnp.int32),
            pltpu.VMEM((RING, CHUNK), jnp.int32),
            pltpu.VMEM((RING, CHUNK), jnp.int32),
            pltpu.VMEM((RING, CHUNK), jnp.int32),
            pltpu.VMEM((RING, CHUNK, DD), jnp.float32),
            pltpu.VMEM_SHARED((ACC_ROWS, DD), jnp.float32),
            pltpu.SemaphoreType.DMA,
            pltpu.SemaphoreType.DMA,
            pltpu.SemaphoreType.DMA,
            pltpu.SemaphoreType.DMA,
            pltpu.SemaphoreType.DMA,
            pltpu.SemaphoreType.DMA,
            pltpu.SemaphoreType.DMA,
            pltpu.SemaphoreType.DMA,
            pltpu.SemaphoreType.DMA,
        ],
    )
    return kern(keys_p, kprev_p, x)


def _mm_body(p_ref, w_ref, b_ref, o_ref):
    xs = p_ref[0] + p_ref[1]
    o_ref[...] = jnp.dot(xs, w_ref[...],
                         preferred_element_type=jnp.float32) + b_ref[...]


@jax.jit
def _combine_matmul(p, w, b2):
    bm = 1000
    return pl.pallas_call(
        _mm_body,
        grid=(NN // bm,),
        in_specs=[
            pl.BlockSpec((NCORES, bm, DD), lambda i: (0, i, 0)),
            pl.BlockSpec((DD, DD), lambda i: (0, 0)),
            pl.BlockSpec((1, DD), lambda i: (0, 0)),
        ],
        out_specs=pl.BlockSpec((bm, DD), lambda i: (i, 0)),
        out_shape=jax.ShapeDtypeStruct((NN, DD), jnp.float32),
    )(p, w, b2)


def kernel(x, edge_list, W, b):
    src = edge_list[0].astype(jnp.int32)
    dst = edge_list[1].astype(jnp.int32)
    keys = src * 16384 + dst
    ks = lax.sort(keys, is_stable=False)
    kprev = jnp.concatenate([jnp.full((1,), -1, jnp.int32), ks[:-1]])
    zpad = jnp.zeros((EPAD - EE,), jnp.int32)
    ksp = jnp.concatenate([ks, zpad])
    kpp = jnp.concatenate([kprev, zpad])
    partials = _sc_segment_sum(ksp, kpp, x)
    return _combine_matmul(partials, W, b.reshape(1, DD))


# speedup vs baseline: 1.0557x; 1.0238x over previous
import jax
import jax.numpy as jnp
from jax import lax
from jax.experimental import pallas as pl
from jax.experimental.pallas import tpu as pltpu
from jax.experimental.pallas import tpu_sc as plsc

NN = 10000
EE = 160000
DD = 128

NCORES = 2
NSUB = 16
NTILES = NCORES * NSUB
CHUNK = 112
NCHUNKS = 45
EDGES_PER_TILE = CHUNK * NCHUNKS
EPAD = NTILES * EDGES_PER_TILE
ACC_ROWS = 10240
ZROWS = ACC_ROWS // NSUB
RING = 3


def _sc_body(keys_hbm, kprev_hbm, x_hbm, p_hbm,
             kv_v, kp_v, tidx_v, sidx_v, rows_v, accum_sh,
             ksem0, ksem1, ksem2, gsem0, gsem1, gsem2, ssem0, ssem1, ssem2):
    c = lax.axis_index("c")
    s = lax.axis_index("s")
    lane = lax.iota(jnp.int32, 16)
    ksems = [ksem0, ksem1, ksem2]
    gsems = [gsem0, gsem1, gsem2]
    ssems = [ssem0, ssem1, ssem2]

    tile_base = (c * NSUB + s) * EDGES_PER_TILE
    trash = 10000 + s

    def start_keys(k):
        b = k % RING
        e0 = tile_base + k * CHUNK
        h1 = pltpu.async_copy(keys_hbm.at[pl.ds(e0, CHUNK)], kv_v.at[b],
                              ksems[b])
        h2 = pltpu.async_copy(kprev_hbm.at[pl.ds(e0, CHUNK)], kp_v.at[b],
                              ksems[b])
        return (h1, h2)

    def compute_indices(k):
        b = k % RING
        def grp(j, carry):
            kv = kv_v[b, pl.ds(j * 16, 16)]
            kp = kp_v[b, pl.ds(j * 16, 16)]
            keep = kv != kp
            src = lax.shift_right_logical(kv, 14)
            t = lax.bitwise_and(kv, 16383)
            dummy_t = lax.bitwise_and(tile_base + k * CHUNK + j * 16 + lane,
                                      8191)
            tidx_v[b, pl.ds(j * 16, 16)] = jnp.where(keep, t, dummy_t)
            sidx_v[b, pl.ds(j * 16, 16)] = jnp.where(keep, src, trash)
            return carry
        lax.fori_loop(0, CHUNK // 16, grp, 0)

    khandles = [None] * NCHUNKS
    for k in range(RING):
        khandles[k] = start_keys(k)

    def zrow(i, carry):
        for j in range(DD // 16):
            rows_v[0, i, pl.ds(j * 16, 16)] = jnp.zeros((16,), jnp.float32)
        return carry
    lax.fori_loop(0, 80, zrow, 0)
    zhandles = []
    for q in range(8):
        zhandles.append(pltpu.async_copy(
            rows_v.at[0, pl.ds(0, 80)],
            accum_sh.at[pl.ds(s * ZROWS + q * 80, 80)], ssems[q % RING]))
    for h in zhandles:
        h.wait()
    plsc.subcore_barrier()

    ghandles = [None] * NCHUNKS
    shandles = [None] * NCHUNKS
    for k in range(NCHUNKS):
        b = k % RING
        for h in khandles[k]:
            h.wait()
        if k >= RING:
            shandles[k - RING].wait()
        compute_indices(k)
        ghandles[k] = pltpu.async_copy(x_hbm.at[tidx_v.at[b]], rows_v.at[b],
                                       gsems[b])
        if k + RING < NCHUNKS:
            khandles[k + RING] = start_keys(k + RING)
        if k >= 1:
            bp = (k - 1) % RING
            ghandles[k - 1].wait()
            shandles[k - 1] = pltpu.async_copy(
                rows_v.at[bp], accum_sh.at[sidx_v.at[bp]], ssems[bp],
                add=True)
    ghandles[NCHUNKS - 1].wait()
    shandles[NCHUNKS - 1] = pltpu.async_copy(
        rows_v.at[(NCHUNKS - 1) % RING],
        accum_sh.at[sidx_v.at[(NCHUNKS - 1) % RING]],
        ssems[(NCHUNKS - 1) % RING], add=True)
    for k in range(NCHUNKS - RING, NCHUNKS):
        shandles[k].wait()
    plsc.subcore_barrier()

    row0 = s * 624
    ehandles = []
    for q in range(8):
        r = row0 + q * 80
        ehandles.append(pltpu.async_copy(
            accum_sh.at[pl.ds(r, 80)], p_hbm.at[c, pl.ds(r, 80)],
            ssems[q % RING]))
    for h in ehandles:
        h.wait()


@jax.jit
def _sc_segment_sum(keys_p, kprev_p, x):
    mesh = plsc.VectorSubcoreMesh(core_axis_name="c", subcore_axis_name="s")
    kern = pl.kernel(
        _sc_body,
        out_type=jax.ShapeDtypeStruct((NCORES, NN, DD), jnp.float32),
        mesh=mesh,
        scratch_types=[
            pltpu.VMEM((RING, CHUNK), jnp.int32),
            pltpu.VMEM((RING, CHUNK), jnp.int32),
            pltpu.VMEM((RING, CHUNK), jnp.int32),
            pltpu.VMEM((RING, CHUNK), jnp.int32),
            pltpu.VMEM((RING, CHUNK, DD), jnp.float32),
            pltpu.VMEM_SHARED((ACC_ROWS, DD), jnp.float32),
            pltpu.SemaphoreType.DMA,
            pltpu.SemaphoreType.DMA,
            pltpu.SemaphoreType.DMA,
            pltpu.SemaphoreType.DMA,
            pltpu.SemaphoreType.DMA,
            pltpu.SemaphoreType.DMA,
            pltpu.SemaphoreType.DMA,
            pltpu.SemaphoreType.DMA,
            pltpu.SemaphoreType.DMA,
        ],
    )
    return kern(keys_p, kprev_p, x)


def _mm_body(p_ref, w_ref, b_ref, o_ref):
    xs = p_ref[0] + p_ref[1]
    o_ref[...] = jnp.dot(xs, w_ref[...],
                         preferred_element_type=jnp.float32) + b_ref[...]


@jax.jit
def _combine_matmul(p, w, b2):
    bm = 1000
    return pl.pallas_call(
        _mm_body,
        grid=(NN // bm,),
        in_specs=[
            pl.BlockSpec((NCORES, bm, DD), lambda i: (0, i, 0)),
            pl.BlockSpec((DD, DD), lambda i: (0, 0)),
            pl.BlockSpec((1, DD), lambda i: (0, 0)),
        ],
        out_specs=pl.BlockSpec((bm, DD), lambda i: (i, 0)),
        out_shape=jax.ShapeDtypeStruct((NN, DD), jnp.float32),
    )(p, w, b2)


def kernel(x, edge_list, W, b):
    src = edge_list[0].astype(jnp.int32)
    dst = edge_list[1].astype(jnp.int32)
    keys = src * 16384 + dst
    ks = lax.sort(keys, is_stable=False)
    kprev = jnp.concatenate([jnp.full((1,), -1, jnp.int32), ks[:-1]])
    zpad = jnp.zeros((EPAD - EE,), jnp.int32)
    ksp = jnp.concatenate([ks, zpad])
    kpp = jnp.concatenate([kprev, zpad])
    partials = _sc_segment_sum(ksp, kpp, x)
    return _combine_matmul(partials, W, b.reshape(1, DD))
